# Initial kernel scaffold; baseline (speedup 1.0000x reference)
import jax
import jax.numpy as jnp
from jax.experimental import pallas as pl


def kernel(input, weight):
    raise NotImplementedError("write your pallas kernel here")


# speedup vs baseline: 1.5480x; 1.5480x over previous
import functools

import jax
import jax.numpy as jnp
from jax import lax
from jax.experimental import pallas as pl
from jax.experimental.pallas import tpu as pltpu
from jax.experimental.pallas import tpu_sc as plsc

VOCAB = 1000000
EMBED = 32
BATCH = 16384
N_FIELDS = 26
B = BATCH * N_FIELDS

NC = 2
NS = 16
NW = NC * NS

B_PER_W = B // NW
CHUNK = 1024
N_CHUNKS = B_PER_W // CHUNK


def _make_gather():
    mesh = plsc.VectorSubcoreMesh(core_axis_name="c", subcore_axis_name="s")

    @functools.partial(
        pl.kernel,
        mesh=mesh,
        out_type=jax.ShapeDtypeStruct((B, EMBED), jnp.float32),
        scratch_types=[
            pltpu.VMEM((CHUNK,), jnp.int32),
            pltpu.VMEM((CHUNK, EMBED), jnp.float32),
            pltpu.SemaphoreType.DMA,
        ],
        compiler_params=pltpu.CompilerParams(use_tc_tiling_on_sc=False),
    )
    def gather_kernel(idx_hbm, w_hbm, out_hbm, idx_v, rows_v, sem):
        wid = lax.axis_index("s") * NC + lax.axis_index("c")
        base = wid * B_PER_W

        @pl.loop(0, N_CHUNKS)
        def _(i):
            off = base + i * CHUNK
            pltpu.sync_copy(idx_hbm.at[pl.ds(off, CHUNK)], idx_v)
            pltpu.async_copy(w_hbm.at[idx_v], rows_v, sem).wait()
            pltpu.sync_copy(rows_v, out_hbm.at[pl.ds(off, CHUNK)])

    return gather_kernel


_gather = _make_gather()


@jax.jit
def kernel(input, weight):
    idx = input.reshape(B)
    out = _gather(idx, weight)
    return out.reshape(BATCH, N_FIELDS, EMBED)


# speedup vs baseline: 1.5763x; 1.0183x over previous
import functools

import jax
import jax.numpy as jnp
from jax import lax
from jax.experimental import pallas as pl
from jax.experimental.pallas import tpu as pltpu
from jax.experimental.pallas import tpu_sc as plsc

VOCAB = 1000000
EMBED = 32
BATCH = 16384
N_FIELDS = 26
B = BATCH * N_FIELDS

NC = 2
NS = 16
NW = NC * NS

B_PER_W = B // NW
CHUNK = 1024
N_CHUNKS = B_PER_W // CHUNK
NBUF = 3


def _make_gather():
    mesh = plsc.VectorSubcoreMesh(core_axis_name="c", subcore_axis_name="s")

    @functools.partial(
        pl.kernel,
        mesh=mesh,
        out_type=jax.ShapeDtypeStruct((B, EMBED), jnp.float32),
        scratch_types=[
            pltpu.VMEM((B_PER_W,), jnp.int32),
            pltpu.VMEM((NBUF, CHUNK, EMBED), jnp.float32),
            pltpu.SemaphoreType.DMA,
            pltpu.SemaphoreType.DMA,
        ],
        compiler_params=pltpu.CompilerParams(use_tc_tiling_on_sc=False),
    )
    def gather_kernel(idx_hbm, w_hbm, out_hbm, idx_all, rows_v, gsem, osem):
        wid = lax.axis_index("s") * NC + lax.axis_index("c")
        base = wid * B_PER_W
        pltpu.sync_copy(idx_hbm.at[pl.ds(base, B_PER_W)], idx_all)

        def gather_chunk(g):
            return pltpu.async_copy(
                w_hbm.at[idx_all.at[pl.ds(g * CHUNK, CHUNK)]],
                rows_v.at[g % NBUF],
                gsem,
            )

        gd = [gather_chunk(g) for g in range(NBUF)]
        od = []
        for g in range(N_CHUNKS):
            gd[g].wait()
            od.append(
                pltpu.async_copy(
                    rows_v.at[g % NBUF],
                    out_hbm.at[pl.ds(base + g * CHUNK, CHUNK)],
                    osem,
                )
            )
            ng = g + NBUF
            if ng < N_CHUNKS:
                od[g].wait()
                gd.append(gather_chunk(ng))
        for g in range(max(0, N_CHUNKS - NBUF), N_CHUNKS):
            od[g].wait()

    return gather_kernel


_gather = _make_gather()


@jax.jit
def kernel(input, weight):
    idx = input.reshape(B)
    out = _gather(idx, weight)
    return out.reshape(BATCH, N_FIELDS, EMBED)
